# Initial kernel scaffold; baseline (speedup 1.0000x reference)
import jax
import jax.numpy as jnp
from jax.experimental import pallas as pl


def kernel(ppg, keys, prompt):
    raise NotImplementedError("write your pallas kernel here")


# speedup vs baseline: 34.1475x; 34.1475x over previous
import functools

import jax
import jax.numpy as jnp
from jax.experimental import pallas as pl
from jax.experimental.pallas import tpu as pltpu

K = 8
GLOBAL_COEFF = 0.5
EPS = 1e-8


def _score_topk_body(nsteps, P, ppg_ref, keys_ref, mask_ref, ssum_ref,
                     ent_ref, score_ref):
    i = pl.program_id(0)
    tile_p = keys_ref.shape[0]

    ppg = ppg_ref[...]
    keys = keys_ref[...]
    dot = jax.lax.dot_general(
        ppg, keys, (((1,), (1,)), ((), ())),
        preferred_element_type=jnp.float32,
        precision=jax.lax.Precision.HIGHEST)
    na = jnp.sqrt(jnp.sum(ppg * ppg, axis=1, keepdims=True))
    nb = jnp.sqrt(jnp.sum(keys * keys, axis=1, keepdims=True)).T
    denom = jnp.maximum(na, EPS) * jnp.maximum(nb, EPS)
    score_ref[:, pl.ds(i * tile_p, tile_p)] = 1.0 - dot / denom

    @pl.when(i == nsteps - 1)
    def _finalize():
        score = score_ref[...]
        bz = score.shape[0]
        iota = jax.lax.broadcasted_iota(jnp.int32, (bz, P), 1)

        mx = jnp.max(score, axis=1, keepdims=True)
        ex = jnp.exp(score - mx)
        se = jnp.sum(ex, axis=1, keepdims=True)
        logp = (score - mx) - jnp.log(se)
        probs = ex / se
        ent_ref[0, 0] = -jnp.sum(probs * logp)

        work = score
        mask = jnp.zeros_like(score)
        for _ in range(K):
            mv = jnp.min(work, axis=1, keepdims=True)
            first = jnp.min(jnp.where(work == mv, iota, P), axis=1,
                            keepdims=True)
            sel = iota == first
            mask = jnp.where(sel, 1.0, mask)
            work = jnp.where(sel, jnp.inf, work)
        mask_ref[...] = mask
        ssum_ref[0, 0] = jnp.sum(score * mask)


def _prompt_sum_body(nsteps, mask_ref, prompt_ref, ppg_ref, out_ref, acc_ref):
    i = pl.program_id(0)
    part = jax.lax.dot_general(
        mask_ref[...], prompt_ref[...], (((1,), (0,)), ((), ())),
        preferred_element_type=jnp.float32,
        precision=jax.lax.Precision.HIGHEST)

    @pl.when(i == 0)
    def _init():
        acc_ref[...] = part

    @pl.when(i > 0)
    def _acc():
        acc_ref[...] += part

    @pl.when(i == nsteps - 1)
    def _emit():
        out_ref[...] = ppg_ref[...] + GLOBAL_COEFF * acc_ref[...]


def kernel(ppg, keys, prompt):
    bz, _, d = ppg.shape
    p = keys.shape[0]
    ppg2d = ppg.reshape(bz, d)

    tile_p = 256
    nsteps = p // tile_p

    mask, ssum, ent = pl.pallas_call(
        functools.partial(_score_topk_body, nsteps, p),
        grid=(nsteps,),
        in_specs=[
            pl.BlockSpec((bz, d), lambda i: (0, 0)),
            pl.BlockSpec((tile_p, d), lambda i: (i, 0)),
        ],
        out_specs=[
            pl.BlockSpec((bz, p), lambda i: (0, 0)),
            pl.BlockSpec(memory_space=pltpu.SMEM),
            pl.BlockSpec(memory_space=pltpu.SMEM),
        ],
        out_shape=[
            jax.ShapeDtypeStruct((bz, p), jnp.float32),
            jax.ShapeDtypeStruct((1, 1), jnp.float32),
            jax.ShapeDtypeStruct((1, 1), jnp.float32),
        ],
        scratch_shapes=[pltpu.VMEM((bz, p), jnp.float32)],
    )(ppg2d, keys)

    prompted = pl.pallas_call(
        functools.partial(_prompt_sum_body, nsteps),
        grid=(nsteps,),
        in_specs=[
            pl.BlockSpec((bz, tile_p), lambda i: (0, i)),
            pl.BlockSpec((tile_p, d), lambda i: (i, 0)),
            pl.BlockSpec((bz, d), lambda i: (0, 0)),
        ],
        out_specs=pl.BlockSpec((bz, d), lambda i: (0, 0)),
        out_shape=jax.ShapeDtypeStruct((bz, d), jnp.float32),
        scratch_shapes=[pltpu.VMEM((bz, d), jnp.float32)],
    )(mask, prompt, ppg2d)

    return prompted.reshape(bz, 1, d), ssum[0, 0], ent[0, 0]


# speedup vs baseline: 50.5244x; 1.4796x over previous
import functools

import jax
import jax.numpy as jnp
from jax.experimental import pallas as pl
from jax.experimental.pallas import tpu as pltpu

K = 8
GLOBAL_COEFF = 0.5
EPS = 1e-8


def _dot3(a, b_hi, b_lo, dims):
    a_hi = a.astype(jnp.bfloat16)
    a_lo = (a - a_hi.astype(jnp.float32)).astype(jnp.bfloat16)
    d = functools.partial(jax.lax.dot_general, dimension_numbers=dims,
                          preferred_element_type=jnp.float32)
    return d(a_hi, b_hi) + (d(a_hi, b_lo) + d(a_lo, b_hi))


def _fused_body(nsteps, P, ppg_ref, keys_ref, prompt_ref, out_ref, ssum_ref,
                ent_ref, score_ref, pscr_ref):
    i = pl.program_id(0)
    tile_p = keys_ref.shape[0]
    dims = (((1,), (1,)), ((), ()))

    keys = keys_ref[...]
    k_hi = keys.astype(jnp.bfloat16)
    k_lo = (keys - k_hi.astype(jnp.float32)).astype(jnp.bfloat16)
    ppg = ppg_ref[...]
    dot = _dot3(ppg, k_hi, k_lo, dims)

    na = jnp.sqrt(jnp.sum(ppg * ppg, axis=1, keepdims=True))
    nb = jnp.sqrt(jnp.sum(keys * keys, axis=1, keepdims=True)).T
    denom = jnp.maximum(na, EPS) * jnp.maximum(nb, EPS)
    score_ref[:, pl.ds(i * tile_p, tile_p)] = 1.0 - dot / denom

    pscr_ref[pl.ds(i * tile_p, tile_p), :] = prompt_ref[...].astype(jnp.bfloat16)

    @pl.when(i == nsteps - 1)
    def _finalize():
        score = score_ref[...]
        bz = score.shape[0]
        iota = jax.lax.broadcasted_iota(jnp.int32, (bz, P), 1)

        mx = jnp.max(score, axis=1, keepdims=True)
        ex = jnp.exp(score - mx)
        se = jnp.sum(ex, axis=1, keepdims=True)
        logp = (score - mx) - jnp.log(se)
        ent_ref[0, 0] = -jnp.sum((ex / se) * logp)

        work = score
        mask = jnp.zeros_like(score)
        for _ in range(K):
            mv = jnp.min(work, axis=1, keepdims=True)
            first = jnp.min(jnp.where(work == mv, iota, P), axis=1,
                            keepdims=True)
            sel = iota == first
            mask = jnp.where(sel, 1.0, mask)
            work = jnp.where(sel, jnp.inf, work)
        ssum_ref[0, 0] = jnp.sum(score * mask)

        psum = jax.lax.dot_general(
            mask.astype(jnp.bfloat16), pscr_ref[...], (((1,), (0,)), ((), ())),
            preferred_element_type=jnp.float32)
        out_ref[...] = ppg + GLOBAL_COEFF * psum


def kernel(ppg, keys, prompt):
    bz, _, d = ppg.shape
    p = keys.shape[0]
    ppg2d = ppg.reshape(bz, d)

    tile_p = 256
    nsteps = p // tile_p

    prompted, ssum, ent = pl.pallas_call(
        functools.partial(_fused_body, nsteps, p),
        grid=(nsteps,),
        in_specs=[
            pl.BlockSpec((bz, d), lambda i: (0, 0)),
            pl.BlockSpec((tile_p, d), lambda i: (i, 0)),
            pl.BlockSpec((tile_p, d), lambda i: (i, 0)),
        ],
        out_specs=[
            pl.BlockSpec((bz, d), lambda i: (0, 0)),
            pl.BlockSpec(memory_space=pltpu.SMEM),
            pl.BlockSpec(memory_space=pltpu.SMEM),
        ],
        out_shape=[
            jax.ShapeDtypeStruct((bz, d), jnp.float32),
            jax.ShapeDtypeStruct((1, 1), jnp.float32),
            jax.ShapeDtypeStruct((1, 1), jnp.float32),
        ],
        scratch_shapes=[
            pltpu.VMEM((bz, p), jnp.float32),
            pltpu.VMEM((p, d), jnp.bfloat16),
        ],
    )(ppg2d, keys, prompt)

    return prompted.reshape(bz, 1, d), ssum[0, 0], ent[0, 0]


# speedup vs baseline: 54.8177x; 1.0850x over previous
import functools

import jax
import jax.numpy as jnp
from jax.experimental import pallas as pl
from jax.experimental.pallas import tpu as pltpu

K = 8
GLOBAL_COEFF = 0.5
EPS = 1e-8


def _dot3(a, b_hi, b_lo, dims):
    a_hi = a.astype(jnp.bfloat16)
    a_lo = (a - a_hi.astype(jnp.float32)).astype(jnp.bfloat16)
    d = functools.partial(jax.lax.dot_general, dimension_numbers=dims,
                          preferred_element_type=jnp.float32)
    return d(a_hi, b_hi) + (d(a_hi, b_lo) + d(a_lo, b_hi))


def _fused_body(nsteps, P, ppg_ref, keys_e_ref, keys_o_ref, prompt_e_ref,
                prompt_o_ref, out_ref, ssum_ref, ent_ref, score_ref, pscr_ref):
    i = pl.program_id(0)
    tile_p = keys_e_ref.shape[0]
    dims = (((1,), (1,)), ((), ()))

    ppg = ppg_ref[...]
    na = jnp.sqrt(jnp.sum(ppg * ppg, axis=1, keepdims=True))

    for half, (k_ref, p_ref) in enumerate(
            [(keys_e_ref, prompt_e_ref), (keys_o_ref, prompt_o_ref)]):
        keys = k_ref[...]
        k_hi = keys.astype(jnp.bfloat16)
        k_lo = (keys - k_hi.astype(jnp.float32)).astype(jnp.bfloat16)
        dot = _dot3(ppg, k_hi, k_lo, dims)

        nb = jnp.sqrt(jnp.sum(keys * keys, axis=1, keepdims=True)).T
        denom = jnp.maximum(na, EPS) * jnp.maximum(nb, EPS)
        off = (2 * i + half) * tile_p
        score_ref[:, pl.ds(off, tile_p)] = 1.0 - dot / denom
        pscr_ref[pl.ds(off, tile_p), :] = p_ref[...].astype(jnp.bfloat16)

    @pl.when(i == nsteps - 1)
    def _finalize():
        score = score_ref[...]
        bz = score.shape[0]
        iota = jax.lax.broadcasted_iota(jnp.int32, (bz, P), 1)

        mx = jnp.max(score, axis=1, keepdims=True)
        ex = jnp.exp(score - mx)
        se = jnp.sum(ex, axis=1, keepdims=True)
        logp = (score - mx) - jnp.log(se)
        ent_ref[0, 0] = -jnp.sum((ex / se) * logp)

        work = score
        mask = jnp.zeros_like(score)
        for _ in range(K):
            mv = jnp.min(work, axis=1, keepdims=True)
            first = jnp.min(jnp.where(work == mv, iota, P), axis=1,
                            keepdims=True)
            sel = iota == first
            mask = jnp.where(sel, 1.0, mask)
            work = jnp.where(sel, jnp.inf, work)
        ssum_ref[0, 0] = jnp.sum(score * mask)

        psum = jax.lax.dot_general(
            mask.astype(jnp.bfloat16), pscr_ref[...], (((1,), (0,)), ((), ())),
            preferred_element_type=jnp.float32)
        out_ref[...] = ppg + GLOBAL_COEFF * psum


def kernel(ppg, keys, prompt):
    bz, _, d = ppg.shape
    p = keys.shape[0]
    ppg2d = ppg.reshape(bz, d)

    tile_p = 128
    nsteps = p // (2 * tile_p)

    prompted, ssum, ent = pl.pallas_call(
        functools.partial(_fused_body, nsteps, p),
        grid=(nsteps,),
        in_specs=[
            pl.BlockSpec((bz, d), lambda i: (0, 0)),
            pl.BlockSpec((tile_p, d), lambda i: (2 * i, 0)),
            pl.BlockSpec((tile_p, d), lambda i: (2 * i + 1, 0)),
            pl.BlockSpec((tile_p, d), lambda i: (2 * i, 0)),
            pl.BlockSpec((tile_p, d), lambda i: (2 * i + 1, 0)),
        ],
        out_specs=[
            pl.BlockSpec((bz, d), lambda i: (0, 0)),
            pl.BlockSpec(memory_space=pltpu.SMEM),
            pl.BlockSpec(memory_space=pltpu.SMEM),
        ],
        out_shape=[
            jax.ShapeDtypeStruct((bz, d), jnp.float32),
            jax.ShapeDtypeStruct((1, 1), jnp.float32),
            jax.ShapeDtypeStruct((1, 1), jnp.float32),
        ],
        scratch_shapes=[
            pltpu.VMEM((bz, p), jnp.float32),
            pltpu.VMEM((p, d), jnp.bfloat16),
        ],
    )(ppg2d, keys, keys, prompt, prompt)

    return prompted.reshape(bz, 1, d), ssum[0, 0], ent[0, 0]
